# baseline (speedup 1.0000x reference)
import functools

import jax
import jax.numpy as jnp
from jax import lax
from jax.experimental import pallas as pl
from jax.experimental.pallas import tpu as pltpu
from jax.experimental.pallas import tpu_sc as plsc

NC = 2
NS = 16
L = 16
NW = NC * NS
SUB = 128
CB = 32
NBUF = 3
TC_BLK = 256
NEG = 0.2


def _proj_body(x_ref, w_ref, a_ref, h_ref, ev_ref):
    h = jnp.dot(x_ref[...], w_ref[...], preferred_element_type=jnp.float32)
    h_ref[...] = h
    ev_ref[...] = lax.dot_general(a_ref[...], h, (((1,), (1,)), ((), ())),
                                  preferred_element_type=jnp.float32)


_EV_ROWS = 16


def _comb_proj_body(acc_ref, b_ref, w_ref, a_ref, h_ref, ev_ref, pre_ref):
    pre = acc_ref[...] + b_ref[...]
    pre_ref[...] = pre
    xb = jnp.maximum(pre, 0.0)
    h = jnp.dot(xb, w_ref[...], preferred_element_type=jnp.float32)
    h_ref[...] = h
    ev_ref[...] = lax.dot_general(a_ref[...], h, (((1,), (1,)), ((), ())),
                                  preferred_element_type=jnp.float32)


def _tc_proj(x, W, A8):
    n_pad, d = x.shape
    h = W.shape[1]
    return pl.pallas_call(
        _proj_body,
        grid=(n_pad // TC_BLK,),
        in_specs=[pl.BlockSpec((TC_BLK, d), lambda i: (i, 0)),
                  pl.BlockSpec((d, h), lambda i: (0, 0)),
                  pl.BlockSpec((_EV_ROWS, h), lambda i: (0, 0))],
        out_specs=[pl.BlockSpec((TC_BLK, h), lambda i: (i, 0)),
                   pl.BlockSpec((_EV_ROWS, TC_BLK), lambda i: (0, i))],
        out_shape=[jax.ShapeDtypeStruct((n_pad, h), jnp.float32),
                   jax.ShapeDtypeStruct((_EV_ROWS, n_pad), jnp.float32)],
    )(x, W, A8)


def _tc_comb_proj(acc, b, W, A8):
    n_pad, d = acc.shape
    h = W.shape[1]
    return pl.pallas_call(
        _comb_proj_body,
        grid=(n_pad // TC_BLK,),
        in_specs=[pl.BlockSpec((TC_BLK, d), lambda i: (i, 0)),
                  pl.BlockSpec((1, d), lambda i: (0, 0)),
                  pl.BlockSpec((d, h), lambda i: (0, 0)),
                  pl.BlockSpec((_EV_ROWS, h), lambda i: (0, 0))],
        out_specs=[pl.BlockSpec((TC_BLK, h), lambda i: (i, 0)),
                   pl.BlockSpec((_EV_ROWS, TC_BLK), lambda i: (0, i)),
                   pl.BlockSpec((TC_BLK, d), lambda i: (i, 0))],
        out_shape=[jax.ShapeDtypeStruct((n_pad, h), jnp.float32),
                   jax.ShapeDtypeStruct((_EV_ROWS, n_pad), jnp.float32),
                   jax.ShapeDtypeStruct((n_pad, d), jnp.float32)],
    )(acc, b, W, A8)


@functools.lru_cache(maxsize=None)
def _make_sc_layer(e_rows, n_pad, d):
    rows_a = e_rows // NS
    ept = rows_a * SUB
    nh = n_pad // NC
    npt = nh // NS
    shift = (nh - 1).bit_length()
    assert shift + (n_pad - 1).bit_length() <= 31
    mask_dl = (1 << shift) - 1
    groups = SUB // L

    mesh = plsc.VectorSubcoreMesh(core_axis_name="c", subcore_axis_name="s")

    def body(src_hbm, dst_hbm, h_hbm, ev_hbm, acc_hbm, dscr_hbm,
             loc_v, dst_v, esrc_v, edst_v, denom_v, rows2_v,
             srcb2_v, dstb2_v, alpha2_v, tmp_v, sg, ss, acc_s):
        rows_v = rows2_v.at[0]
        c = lax.axis_index("c")
        s = lax.axis_index("s")
        base = c * nh
        iota = lax.iota(jnp.int32, L)
        zero16 = jnp.zeros((L,), jnp.float32)

        pltpu.sync_copy(src_hbm.at[pl.ds(s * ept, ept)], loc_v.at[pl.ds(0, ept)])
        pltpu.sync_copy(dst_hbm.at[pl.ds(s * ept, ept)], dst_v)
        pltpu.sync_copy(ev_hbm.at[0], esrc_v)
        pltpu.sync_copy(ev_hbm.at[8, pl.ds(base, nh)], edst_v)

        @pl.loop(0, nh // L)
        def _(i):
            denom_v[pl.ds(i * L, L)] = zero16

        @pl.loop(0, CB)
        def _(i):
            for j in range(d // L):
                rows2_v[0, i, pl.ds(j * L, L)] = zero16

        q = 0
        while q + CB <= npt:
            pltpu.sync_copy(rows_v, acc_s.at[pl.ds(s * npt + q, CB)])
            q += CB
        if q < npt:
            pltpu.sync_copy(rows_v.at[pl.ds(0, npt - q)],
                            acc_s.at[pl.ds(s * npt + q, npt - q)])

        @pl.loop(0, ept // L, init_carry=jnp.int32(0))
        def cnt(g, cnt):
            off = g * L
            s16 = loc_v[pl.ds(off, L)]
            dl = dst_v[pl.ds(off, L)] - base
            keep = (dl >= 0) & (dl < nh)
            packed = lax.shift_left(s16, shift) | (dl & mask_dl)
            plsc.store_compressed(loc_v.at[pl.ds(cnt, L)], packed, mask=keep)
            return cnt + jnp.sum(keep.astype(jnp.int32))

        ones = jnp.ones((L,), jnp.bool_)
        for j in range(2 * groups):
            plsc.store_compressed(loc_v.at[pl.ds(cnt + j * L, L)],
                                  jnp.zeros((L,), jnp.int32), mask=ones)

        @pl.loop(0, (cnt + L - 1) // L)
        def _(g):
            off = g * L
            p16 = loc_v[pl.ds(off, L)]
            sl = lax.shift_right_logical(p16, shift)
            dl = p16 & mask_dl
            e = (plsc.load_gather(esrc_v, [sl]) +
                 plsc.load_gather(edst_v, [dl]))
            e = jnp.where(e > 0, e, e * NEG)
            valid = (iota + off) < cnt
            plsc.addupdate_scatter(denom_v, [dl], jnp.exp(e), mask=valid)

        slab = nh // 8
        dbase = c * NS * nh
        pltpu.sync_copy(denom_v, dscr_hbm.at[pl.ds(dbase + s * nh, nh)])
        plsc.subcore_barrier()

        @pl.when(s < 8)
        def _():
            for t in range(NS):
                pltpu.sync_copy(
                    dscr_hbm.at[pl.ds(dbase + t * nh + s * slab, slab)],
                    tmp_v.at[pl.ds(t * slab, slab)])

            @pl.loop(0, slab // L)
            def _(j):
                tot = tmp_v[pl.ds(j * L, L)]
                for t in range(1, NS):
                    tot = tot + tmp_v[pl.ds(t * slab + j * L, L)]
                tmp_v[pl.ds(j * L, L)] = tot

            pltpu.sync_copy(tmp_v.at[pl.ds(0, slab)],
                            dscr_hbm.at[pl.ds(dbase + s * slab, slab)])

        plsc.subcore_barrier()
        pltpu.sync_copy(dscr_hbm.at[pl.ds(dbase, nh)], denom_v)

        nm = (cnt + CB - 1) // CB

        def build(m, p):
            off = m * CB
            for g in range(CB // L):
                o = off + g * L
                p16 = loc_v[pl.ds(o, L)]
                sl = lax.shift_right_logical(p16, shift)
                dl = p16 & mask_dl
                srcb2_v[p, pl.ds(g * L, L)] = sl
                dstb2_v[p, pl.ds(g * L, L)] = dl
                e = (plsc.load_gather(esrc_v, [sl]) +
                     plsc.load_gather(edst_v, [dl]))
                e = jnp.where(e > 0, e, e * NEG)
                den = plsc.load_gather(denom_v, [dl])
                a = jnp.exp(e) / (den + 1e-16)
                valid = (iota + o) < cnt
                alpha2_v[p, pl.ds(g * L, L)] = jnp.where(valid, a, 0.0)

        def fire(p):
            pltpu.async_copy(h_hbm.at[srcb2_v.at[p]], rows2_v.at[p],
                             sg.at[p])

        def wait_scatter(p):
            pltpu.make_async_copy(rows2_v.at[p], acc_s.at[dstb2_v.at[p]],
                                  ss.at[p]).wait()

        for q in range(NBUF - 1):
            @pl.when(q < nm)
            def _():
                build(q, q)
                fire(q)

        @pl.loop(0, nm)
        def _(m):
            f = m + NBUF - 1

            @pl.when(f < nm)
            def _():
                fp = f % NBUF

                @pl.when(m >= 1)
                def _():
                    wait_scatter(fp)

                build(f, fp)
                fire(fp)

            p = m % NBUF
            pltpu.make_async_copy(h_hbm.at[srcb2_v.at[p]],
                                  rows2_v.at[p], sg.at[p]).wait()

            @pl.loop(0, CB, unroll=2)
            def _(i):
                a = plsc.load_gather(alpha2_v.at[p],
                                     [jnp.zeros((L,), jnp.int32) + i])
                for j in range(d // L):
                    rows2_v[p, i, pl.ds(j * L, L)] = (
                        rows2_v[p, i, pl.ds(j * L, L)] * a)

            pltpu.async_copy(rows2_v.at[p], acc_s.at[dstb2_v.at[p]],
                             ss.at[p], add=True)

        for q in range(NBUF):
            @pl.when(q < nm)
            def _():
                wait_scatter((nm - 1 - q) % NBUF)

        plsc.subcore_barrier()
        q = 0
        while q + SUB <= npt:
            pltpu.sync_copy(acc_s.at[pl.ds(s * npt + q, SUB)],
                            acc_hbm.at[pl.ds(base + s * npt + q, SUB)])
            q += SUB
        if q < npt:
            pltpu.sync_copy(acc_s.at[pl.ds(s * npt + q, npt - q)],
                            acc_hbm.at[pl.ds(base + s * npt + q, npt - q)])

    return pl.kernel(
        body,
        out_type=(jax.ShapeDtypeStruct((n_pad, d), jnp.float32),
                  jax.ShapeDtypeStruct((NC * NS * nh,), jnp.float32)),
        mesh=mesh,
        compiler_params=pltpu.CompilerParams(needs_layout_passes=False),
        scratch_types=[
            pltpu.VMEM((ept + 2 * SUB,), jnp.int32),
            pltpu.VMEM((ept,), jnp.int32),
            pltpu.VMEM((n_pad,), jnp.float32),
            pltpu.VMEM((nh,), jnp.float32),
            pltpu.VMEM((nh,), jnp.float32),
            pltpu.VMEM((NBUF, CB, d), jnp.float32),
            pltpu.VMEM((NBUF, CB), jnp.int32),
            pltpu.VMEM((NBUF, CB), jnp.int32),
            pltpu.VMEM((NBUF, CB), jnp.float32),
            pltpu.VMEM((NS * (nh // 8),), jnp.float32),
            pltpu.SemaphoreType.DMA((NBUF,)),
            pltpu.SemaphoreType.DMA((NBUF,)),
            pltpu.VMEM_SHARED((nh, d), jnp.float32),
        ],
    )


def kernel(x, edge_index, W1, a_src1, a_dst1, b1, W2, a_src2, a_dst2, b2):
    n, d = x.shape
    e = edge_index.shape[1]
    n_pad = ((n + 1 + 2047) // 2048) * 2048
    etot = e + n
    e_pad = ((etot + NW * SUB - 1) // (NW * SUB)) * (NW * SUB)

    loop_idx = jnp.arange(n, dtype=jnp.int32)
    src = jnp.concatenate([edge_index[0].astype(jnp.int32), loop_idx,
                           jnp.zeros((e_pad - etot,), jnp.int32)])
    dst = jnp.concatenate([edge_index[1].astype(jnp.int32), loop_idx,
                           jnp.full((e_pad - etot,), n, jnp.int32)])
    src2 = src
    dst2 = dst

    xp = jnp.zeros((n_pad, d), jnp.float32).at[:n].set(x)
    A1 = (jnp.zeros((_EV_ROWS, W1.shape[1]), jnp.float32)
          .at[0].set(a_src1).at[8].set(a_dst1))
    A2 = (jnp.zeros((_EV_ROWS, W2.shape[1]), jnp.float32)
          .at[0].set(a_src2).at[8].set(a_dst2))

    sc_layer = _make_sc_layer(e_pad // SUB, n_pad, W1.shape[1])

    h1, ev1 = _tc_proj(xp, W1, A1)

    Wx = jnp.stack([W2, W2])
    Ax = jnp.stack([A2, A2])
    bx = jnp.stack([b1.reshape(1, -1), b2.reshape(1, -1)])

    def step(carry, wab):
        h, ev = carry
        W_i, A_i, b_i = wab
        acc, _ = sc_layer(src2, dst2, h, ev)
        hn, evn, pre = _tc_comb_proj(acc, b_i, W_i, A_i)
        return (hn, evn), pre

    _, pres = lax.scan(step, (h1, ev1), (Wx, Ax, bx))
    return pres[1][:n]


# speedup vs baseline: 1.6406x; 1.6406x over previous
import functools

import jax
import jax.numpy as jnp
from jax import lax
from jax.experimental import pallas as pl
from jax.experimental.pallas import tpu as pltpu
from jax.experimental.pallas import tpu_sc as plsc

NC = 2
NS = 16
L = 16
NW = NC * NS
SUB = 128
CB = 64
TC_BLK = 256
NEG = 0.2


def _proj_body(x_ref, w_ref, a_ref, h_ref, ev_ref):
    h = jnp.dot(x_ref[...], w_ref[...], preferred_element_type=jnp.float32)
    h_ref[...] = h
    ev_ref[...] = lax.dot_general(a_ref[...], h, (((1,), (1,)), ((), ())),
                                  preferred_element_type=jnp.float32)


_EV_ROWS = 16


def _comb_proj_body(acc_ref, b_ref, w_ref, a_ref, h_ref, ev_ref, pre_ref):
    pre = acc_ref[...] + b_ref[...]
    pre_ref[...] = pre
    xb = jnp.maximum(pre, 0.0)
    h = jnp.dot(xb, w_ref[...], preferred_element_type=jnp.float32)
    h_ref[...] = h
    ev_ref[...] = lax.dot_general(a_ref[...], h, (((1,), (1,)), ((), ())),
                                  preferred_element_type=jnp.float32)


def _tc_proj(x, W, A8):
    n_pad, d = x.shape
    h = W.shape[1]
    return pl.pallas_call(
        _proj_body,
        grid=(n_pad // TC_BLK,),
        in_specs=[pl.BlockSpec((TC_BLK, d), lambda i: (i, 0)),
                  pl.BlockSpec((d, h), lambda i: (0, 0)),
                  pl.BlockSpec((_EV_ROWS, h), lambda i: (0, 0))],
        out_specs=[pl.BlockSpec((TC_BLK, h), lambda i: (i, 0)),
                   pl.BlockSpec((_EV_ROWS, TC_BLK), lambda i: (0, i))],
        out_shape=[jax.ShapeDtypeStruct((n_pad, h), jnp.float32),
                   jax.ShapeDtypeStruct((_EV_ROWS, n_pad), jnp.float32)],
    )(x, W, A8)


def _tc_comb_proj(acc, b, W, A8):
    n_pad, d = acc.shape
    h = W.shape[1]
    return pl.pallas_call(
        _comb_proj_body,
        grid=(n_pad // TC_BLK,),
        in_specs=[pl.BlockSpec((TC_BLK, d), lambda i: (i, 0)),
                  pl.BlockSpec((1, d), lambda i: (0, 0)),
                  pl.BlockSpec((d, h), lambda i: (0, 0)),
                  pl.BlockSpec((_EV_ROWS, h), lambda i: (0, 0))],
        out_specs=[pl.BlockSpec((TC_BLK, h), lambda i: (i, 0)),
                   pl.BlockSpec((_EV_ROWS, TC_BLK), lambda i: (0, i)),
                   pl.BlockSpec((TC_BLK, d), lambda i: (i, 0))],
        out_shape=[jax.ShapeDtypeStruct((n_pad, h), jnp.float32),
                   jax.ShapeDtypeStruct((_EV_ROWS, n_pad), jnp.float32),
                   jax.ShapeDtypeStruct((n_pad, d), jnp.float32)],
    )(acc, b, W, A8)


@functools.lru_cache(maxsize=None)
def _make_sc_layer(e_rows, n_pad, d):
    rows_a = e_rows // NS
    ept = rows_a * SUB
    nh = n_pad // NC
    npt = nh // NS
    shift = (nh - 1).bit_length()
    assert shift + (n_pad - 1).bit_length() <= 31
    mask_dl = (1 << shift) - 1
    groups = SUB // L

    mesh = plsc.VectorSubcoreMesh(core_axis_name="c", subcore_axis_name="s")

    def body(src_hbm, dst_hbm, h_hbm, ev_hbm, acc_hbm, dscr_hbm,
             loc_v, dst_v, esrc_v, edst_v, denom_v, rows2_v,
             srcb2_v, dstb2_v, alpha2_v, tmp_v, sg, ss, acc_s):
        rows_v = rows2_v.at[0]
        c = lax.axis_index("c")
        s = lax.axis_index("s")
        base = c * nh
        iota = lax.iota(jnp.int32, L)
        zero16 = jnp.zeros((L,), jnp.float32)

        pltpu.async_copy(ev_hbm.at[0], esrc_v, sg.at[0])
        pltpu.async_copy(ev_hbm.at[8, pl.ds(base, nh)], edst_v, sg.at[1])
        pltpu.sync_copy(src_hbm.at[pl.ds(s * ept, ept)], loc_v.at[pl.ds(0, ept)])
        pltpu.sync_copy(dst_hbm.at[pl.ds(s * ept, ept)], dst_v)

        @pl.loop(0, nh // L)
        def _(i):
            denom_v[pl.ds(i * L, L)] = zero16

        @pl.loop(0, CB)
        def _(i):
            for j in range(d // L):
                rows2_v[0, i, pl.ds(j * L, L)] = zero16

        q = 0
        while q + CB <= npt:
            pltpu.sync_copy(rows_v, acc_s.at[pl.ds(s * npt + q, CB)])
            q += CB
        if q < npt:
            pltpu.sync_copy(rows_v.at[pl.ds(0, npt - q)],
                            acc_s.at[pl.ds(s * npt + q, npt - q)])

        @pl.loop(0, ept // L, init_carry=jnp.int32(0))
        def cnt(g, cnt):
            off = g * L
            s16 = loc_v[pl.ds(off, L)]
            dl = dst_v[pl.ds(off, L)] - base
            keep = (dl >= 0) & (dl < nh)
            packed = lax.shift_left(s16, shift) | (dl & mask_dl)
            plsc.store_compressed(loc_v.at[pl.ds(cnt, L)], packed, mask=keep)
            return cnt + jnp.sum(keep.astype(jnp.int32))

        ones = jnp.ones((L,), jnp.bool_)
        for j in range(2 * groups):
            plsc.store_compressed(loc_v.at[pl.ds(cnt + j * L, L)],
                                  jnp.zeros((L,), jnp.int32), mask=ones)

        pltpu.make_async_copy(ev_hbm.at[0], esrc_v, sg.at[0]).wait()
        pltpu.make_async_copy(ev_hbm.at[8, pl.ds(base, nh)], edst_v,
                              sg.at[1]).wait()

        @pl.loop(0, (cnt + L - 1) // L)
        def _(g):
            off = g * L
            p16 = loc_v[pl.ds(off, L)]
            sl = lax.shift_right_logical(p16, shift)
            dl = p16 & mask_dl
            e = (plsc.load_gather(esrc_v, [sl]) +
                 plsc.load_gather(edst_v, [dl]))
            e = jnp.where(e > 0, e, e * NEG)
            valid = (iota + off) < cnt
            plsc.addupdate_scatter(denom_v, [dl], jnp.exp(e), mask=valid)

        slab = nh // 8
        dbase = c * NS * nh
        pltpu.sync_copy(denom_v, dscr_hbm.at[pl.ds(dbase + s * nh, nh)])
        plsc.subcore_barrier()

        @pl.when(s < 8)
        def _():
            for t in range(NS):
                pltpu.sync_copy(
                    dscr_hbm.at[pl.ds(dbase + t * nh + s * slab, slab)],
                    tmp_v.at[pl.ds(t * slab, slab)])

            @pl.loop(0, slab // L)
            def _(j):
                tot = tmp_v[pl.ds(j * L, L)]
                for t in range(1, NS):
                    tot = tot + tmp_v[pl.ds(t * slab + j * L, L)]
                tmp_v[pl.ds(j * L, L)] = tot

            pltpu.sync_copy(tmp_v.at[pl.ds(0, slab)],
                            dscr_hbm.at[pl.ds(dbase + s * slab, slab)])

        plsc.subcore_barrier()
        pltpu.sync_copy(dscr_hbm.at[pl.ds(dbase, nh)], denom_v)

        nm = (cnt + CB - 1) // CB

        def build(m, p):
            off = m * CB
            for g in range(CB // L):
                o = off + g * L
                p16 = loc_v[pl.ds(o, L)]
                sl = lax.shift_right_logical(p16, shift)
                dl = p16 & mask_dl
                srcb2_v[p, pl.ds(g * L, L)] = sl
                dstb2_v[p, pl.ds(g * L, L)] = dl
                e = (plsc.load_gather(esrc_v, [sl]) +
                     plsc.load_gather(edst_v, [dl]))
                e = jnp.where(e > 0, e, e * NEG)
                den = plsc.load_gather(denom_v, [dl])
                a = jnp.exp(e) / (den + 1e-16)
                valid = (iota + o) < cnt
                alpha2_v[p, pl.ds(g * L, L)] = jnp.where(valid, a, 0.0)

        def fire(p):
            pltpu.async_copy(h_hbm.at[srcb2_v.at[p]], rows2_v.at[p],
                             sg.at[p])

        def wait_scatter(p):
            pltpu.make_async_copy(rows2_v.at[p], acc_s.at[dstb2_v.at[p]],
                                  ss.at[p]).wait()

        @pl.when(nm > 0)
        def _():
            build(0, 0)
            fire(0)

        @pl.loop(0, nm)
        def _(m):
            p = m & 1

            @pl.when(m + 1 < nm)
            def _():
                @pl.when(m >= 1)
                def _():
                    wait_scatter(1 - p)

                build(m + 1, 1 - p)
                fire(1 - p)

            pltpu.make_async_copy(h_hbm.at[srcb2_v.at[p]],
                                  rows2_v.at[p], sg.at[p]).wait()

            @pl.loop(0, CB, unroll=4)
            def _(i):
                a = plsc.load_gather(alpha2_v.at[p],
                                     [jnp.zeros((L,), jnp.int32) + i])
                for j in range(d // L):
                    rows2_v[p, i, pl.ds(j * L, L)] = (
                        rows2_v[p, i, pl.ds(j * L, L)] * a)

            pltpu.async_copy(rows2_v.at[p], acc_s.at[dstb2_v.at[p]],
                             ss.at[p], add=True)

        @pl.when(nm > 0)
        def _():
            wait_scatter((nm - 1) & 1)

        @pl.when(nm > 1)
        def _():
            wait_scatter(nm & 1)

        plsc.subcore_barrier()
        q = 0
        while q + SUB <= npt:
            pltpu.sync_copy(acc_s.at[pl.ds(s * npt + q, SUB)],
                            acc_hbm.at[pl.ds(base + s * npt + q, SUB)])
            q += SUB
        if q < npt:
            pltpu.sync_copy(acc_s.at[pl.ds(s * npt + q, npt - q)],
                            acc_hbm.at[pl.ds(base + s * npt + q, npt - q)])

    return pl.kernel(
        body,
        out_type=(jax.ShapeDtypeStruct((n_pad, d), jnp.float32),
                  jax.ShapeDtypeStruct((NC * NS * nh,), jnp.float32)),
        mesh=mesh,
        compiler_params=pltpu.CompilerParams(needs_layout_passes=False),
        scratch_types=[
            pltpu.VMEM((ept + 2 * SUB,), jnp.int32),
            pltpu.VMEM((ept,), jnp.int32),
            pltpu.VMEM((n_pad,), jnp.float32),
            pltpu.VMEM((nh,), jnp.float32),
            pltpu.VMEM((nh,), jnp.float32),
            pltpu.VMEM((2, CB, d), jnp.float32),
            pltpu.VMEM((2, CB), jnp.int32),
            pltpu.VMEM((2, CB), jnp.int32),
            pltpu.VMEM((2, CB), jnp.float32),
            pltpu.VMEM((NS * (nh // 8),), jnp.float32),
            pltpu.SemaphoreType.DMA((2,)),
            pltpu.SemaphoreType.DMA((2,)),
            pltpu.VMEM_SHARED((nh, d), jnp.float32),
        ],
    )


def kernel(x, edge_index, W1, a_src1, a_dst1, b1, W2, a_src2, a_dst2, b2):
    n, d = x.shape
    e = edge_index.shape[1]
    n_pad = ((n + 1 + 2047) // 2048) * 2048
    etot = e + n
    e_pad = ((etot + NW * SUB - 1) // (NW * SUB)) * (NW * SUB)

    loop_idx = jnp.arange(n, dtype=jnp.int32)
    src = jnp.concatenate([edge_index[0].astype(jnp.int32), loop_idx,
                           jnp.zeros((e_pad - etot,), jnp.int32)])
    dst = jnp.concatenate([edge_index[1].astype(jnp.int32), loop_idx,
                           jnp.full((e_pad - etot,), n, jnp.int32)])
    src2 = src
    dst2 = dst

    xp = jnp.zeros((n_pad, d), jnp.float32).at[:n].set(x)
    A1 = (jnp.zeros((_EV_ROWS, W1.shape[1]), jnp.float32)
          .at[0].set(a_src1).at[8].set(a_dst1))
    A2 = (jnp.zeros((_EV_ROWS, W2.shape[1]), jnp.float32)
          .at[0].set(a_src2).at[8].set(a_dst2))

    sc_layer = _make_sc_layer(e_pad // SUB, n_pad, W1.shape[1])

    h1, ev1 = _tc_proj(xp, W1, A1)

    Wx = jnp.stack([W2, W2])
    Ax = jnp.stack([A2, A2])
    bx = jnp.stack([b1.reshape(1, -1), b2.reshape(1, -1)])

    def step(carry, wab):
        h, ev = carry
        W_i, A_i, b_i = wab
        acc, _ = sc_layer(src2, dst2, h, ev)
        hn, evn, pre = _tc_comb_proj(acc, b_i, W_i, A_i)
        return (hn, evn), pre

    _, pres = lax.scan(step, (h1, ev1), (Wx, Ax, bx))
    return pres[1][:n]
